# baseline (speedup 1.0000x reference)
import functools

import jax
import jax.numpy as jnp
from jax.experimental import pallas as pl
from jax.experimental.pallas import tpu as pltpu


def _round_up(x, m):
    return ((x + m - 1) // m) * m


def _fused_kernel(idx_smem, tgt_ref, table_ref, logits_ref, loss_ref,
                  tg_ref, scratch_ref, acc_ref, *, n_tokens, v_vocab,
                  tile_n, tile_v, n_prep, n_tiles, n_aug, stride):
    i = pl.program_id(0)
    tn, vp = logits_ref.shape
    n_chunk = vp // 128
    n_slab = n_chunk + 1

    @pl.when(i < n_prep)
    def _prep():
        t = table_ref[...]
        if v_vocab < vp:
            lane = jax.lax.broadcasted_iota(jnp.int32, (tile_v, vp), 1)
            t = jnp.where(lane < v_vocab, t, jnp.float32(-1e30))
        m = jnp.max(t, axis=-1, keepdims=True)
        p = jnp.exp(t - m)
        if v_vocab < vp:
            p = jnp.where(lane < v_vocab, p, 0.0)
            t = table_ref[...]
        lse = m + jnp.log(jnp.sum(p, axis=-1, keepdims=True))
        base = i * tile_v * n_aug
        for c in range(n_chunk):
            tg_ref[pl.ds(base + c, tile_v, n_aug), :] = \
                t[:, c * 128:(c + 1) * 128]
        tg_ref[pl.ds(base + n_chunk, tile_v, n_aug), :] = \
            jnp.broadcast_to(lse, (tile_v, 128))

    @pl.when(i >= n_prep)
    def _gather():
        it = i - n_prep
        tok0 = it * tile_n
        for mi in range(tile_n):
            base = pl.multiple_of(idx_smem[tok0 + mi] * n_aug, 8)
            slab = tg_ref[pl.ds(base, n_slab), :]
            scratch_ref[pl.ds(mi, n_slab, stride), :] = slab

        tgt = tgt_ref[...]
        lane = jax.lax.broadcasted_iota(jnp.int32, (tn, 128), 1)
        picked = jnp.zeros((tn, 1), jnp.float32)
        for c in range(n_chunk):
            plane = scratch_ref[pl.ds(c * stride, tn), :]
            logits_ref[:, c * 128:(c + 1) * 128] = plane
            hit = (lane + c * 128) == tgt
            picked = picked + jnp.sum(jnp.where(hit, plane, 0.0),
                                      axis=-1, keepdims=True)

        lse = scratch_ref[pl.ds(n_chunk * stride, tn), 0:1]
        row = jax.lax.broadcasted_iota(jnp.int32, (tn, 1), 0) + tok0
        per_row = jnp.where(row < n_tokens, lse - picked, 0.0)
        part = jnp.sum(per_row)
        prev = jnp.where(it == 0, 0.0, acc_ref[0])
        total = prev + part
        acc_ref[0] = total

        @pl.when(it == n_tiles - 1)
        def _emit():
            loss_ref[...] = jnp.full(loss_ref.shape, total, jnp.float32)


def kernel(idx, table, targets):
    idx = jnp.asarray(idx, jnp.int32)
    table = jnp.asarray(table, jnp.float32)
    B, T = idx.shape
    V = table.shape[0]
    N = B * T

    v_pad = _round_up(V, 128)
    n_chunk = v_pad // 128
    n_aug = _round_up(n_chunk + 1, 8)
    tile_n = min(512, _round_up(N, 8))
    num_tiles = pl.cdiv(N, tile_n)
    n_pad = num_tiles * tile_n
    stride = tile_n + 8
    scratch_rows = stride * n_chunk + tile_n
    tile_v = 256 if v_pad % 256 == 0 else 128
    n_prep = v_pad // tile_v

    table_p = table if v_pad == V else jnp.pad(
        table, ((0, v_pad - V), (0, v_pad - V)))

    idx_p = jnp.pad(idx.reshape(-1), (0, n_pad - N))
    tgt_p = jnp.pad(jnp.asarray(targets, jnp.int32).reshape(-1),
                    (0, n_pad - N)).reshape(n_pad, 1)

    gi = lambda i: jnp.maximum(i - n_prep, 0)
    logits_p, loss_part = pl.pallas_call(
        functools.partial(_fused_kernel, n_tokens=N, v_vocab=V,
                          tile_n=tile_n, tile_v=tile_v, n_prep=n_prep,
                          n_tiles=num_tiles, n_aug=n_aug, stride=stride),
        out_shape=(
            jax.ShapeDtypeStruct((n_pad, v_pad), jnp.float32),
            jax.ShapeDtypeStruct((1, 1, 128), jnp.float32),
        ),
        grid=(n_prep + num_tiles,),
        in_specs=[
            pl.BlockSpec(memory_space=pltpu.SMEM),
            pl.BlockSpec((tile_n, 1), lambda i: (gi(i), 0)),
            pl.BlockSpec((tile_v, v_pad),
                         lambda i: (jnp.minimum(i, n_prep - 1), 0)),
        ],
        out_specs=(
            pl.BlockSpec((tile_n, v_pad), lambda i: (gi(i), 0)),
            pl.BlockSpec((1, 1, 128), lambda i: (0, 0, 0)),
        ),
        scratch_shapes=[
            pltpu.VMEM((v_pad * n_aug, 128), jnp.float32),
            pltpu.VMEM((scratch_rows, 128), jnp.float32),
            pltpu.SMEM((1,), jnp.float32),
        ],
        compiler_params=pltpu.CompilerParams(
            dimension_semantics=("arbitrary",)),
    )(idx_p, tgt_p, table_p)

    loss = loss_part[0, 0, 0] / N
    logits = logits_p[:N, :V]
    return logits, loss


# speedup vs baseline: 1.5141x; 1.5141x over previous
import functools

import jax
import jax.numpy as jnp
from jax.experimental import pallas as pl
from jax.experimental.pallas import tpu as pltpu


def _round_up(x, m):
    return ((x + m - 1) // m) * m


def _fused_kernel(idx_smem, tgt_ref, table_ref, logits_ref, loss_ref,
                  tg_ref, scratch_ref, *, n_tokens, v_vocab, tile_n, tile_v,
                  n_prep, n_aug, stride):
    i = pl.program_id(0)
    tn, vp = logits_ref.shape
    n_chunk = vp // 128
    n_slab = n_chunk + 1

    @pl.when(i < n_prep)
    def _prep():
        t = table_ref[...]
        if v_vocab < vp:
            lane = jax.lax.broadcasted_iota(jnp.int32, (tile_v, vp), 1)
            t = jnp.where(lane < v_vocab, t, jnp.float32(-1e30))
        m = jnp.max(t, axis=-1, keepdims=True)
        p = jnp.exp(t - m)
        if v_vocab < vp:
            p = jnp.where(lane < v_vocab, p, 0.0)
            t = table_ref[...]
        lse = m + jnp.log(jnp.sum(p, axis=-1, keepdims=True))
        base = i * tile_v * n_aug
        for c in range(n_chunk):
            tg_ref[pl.ds(base + c, tile_v, n_aug), :] = \
                t[:, c * 128:(c + 1) * 128]
        tg_ref[pl.ds(base + n_chunk, tile_v, n_aug), :] = \
            jnp.broadcast_to(lse, (tile_v, 128))

    @pl.when(i >= n_prep)
    def _gather():
        for mi in range(tile_n):
            base = pl.multiple_of(idx_smem[mi] * n_aug, 8)
            slab = tg_ref[pl.ds(base, n_slab), :]
            scratch_ref[pl.ds(mi * n_aug, n_slab), :] = slab

        planes = [scratch_ref[pl.ds(c, tn, n_aug), :]
                  for c in range(n_chunk)]
        logits = jnp.concatenate(planes, axis=1)
        logits_ref[...] = logits

        lse = scratch_ref[pl.ds(n_chunk, tn, n_aug), 0:1]
        tgt = tgt_ref[...]
        col = jax.lax.broadcasted_iota(jnp.int32, (tn, vp), 1)
        picked = jnp.sum(jnp.where(col == tgt, logits, 0.0),
                         axis=-1, keepdims=True)
        row = (jax.lax.broadcasted_iota(jnp.int32, (tn, 1), 0)
               + (i - n_prep) * tile_n)
        per_row = jnp.where(row < n_tokens, lse - picked, 0.0)
        loss_ref[...] = jnp.full(loss_ref.shape, jnp.sum(per_row),
                                 jnp.float32)


def kernel(idx, table, targets):
    idx = jnp.asarray(idx, jnp.int32)
    table = jnp.asarray(table, jnp.float32)
    B, T = idx.shape
    V = table.shape[0]
    N = B * T

    v_pad = _round_up(V, 128)
    n_chunk = v_pad // 128
    n_aug = _round_up(n_chunk + 1, 8)
    tile_n = min(512, _round_up(N, 8))
    num_tiles = pl.cdiv(N, tile_n)
    n_pad = num_tiles * tile_n
    stride = tile_n + 8
    scratch_rows = tile_n * n_aug
    tile_v = 256 if v_pad % 256 == 0 else 128
    n_prep = v_pad // tile_v

    table_p = table if v_pad == V else jnp.pad(
        table, ((0, v_pad - V), (0, v_pad - V)))

    idx_p = jnp.pad(idx.reshape(-1), (0, n_pad - N))
    tgt_p = jnp.pad(jnp.asarray(targets, jnp.int32).reshape(-1),
                    (0, n_pad - N)).reshape(n_pad, 1)

    gi = lambda i: jnp.maximum(i - n_prep, 0)
    logits_p, loss_part = pl.pallas_call(
        functools.partial(_fused_kernel, n_tokens=N, v_vocab=V,
                          tile_n=tile_n, tile_v=tile_v, n_prep=n_prep,
                          n_aug=n_aug, stride=stride),
        out_shape=(
            jax.ShapeDtypeStruct((n_pad, v_pad), jnp.float32),
            jax.ShapeDtypeStruct((num_tiles, 1, 128), jnp.float32),
        ),
        grid=(n_prep + num_tiles,),
        in_specs=[
            pl.BlockSpec((tile_n,), lambda i: (gi(i),),
                         memory_space=pltpu.SMEM),
            pl.BlockSpec((tile_n, 1), lambda i: (gi(i), 0)),
            pl.BlockSpec((tile_v, v_pad),
                         lambda i: (jnp.minimum(i, n_prep - 1), 0)),
        ],
        out_specs=(
            pl.BlockSpec((tile_n, v_pad), lambda i: (gi(i), 0)),
            pl.BlockSpec((1, 1, 128), lambda i: (gi(i), 0, 0)),
        ),
        scratch_shapes=[
            pltpu.VMEM((v_pad * n_aug, 128), jnp.float32),
            pltpu.VMEM((scratch_rows, 128), jnp.float32),
        ],
        compiler_params=pltpu.CompilerParams(
            dimension_semantics=("arbitrary",)),
    )(idx_p, tgt_p, table_p)

    loss = loss_part[:, 0, 0].sum() / N
    logits = logits_p[:N, :V]
    return logits, loss
